# Initial kernel scaffold; baseline (speedup 1.0000x reference)
import jax
import jax.numpy as jnp
from jax.experimental import pallas as pl


def kernel(image, feat, rpn_conv_w, rpn_conv_b, cls_w, cls_b, reg_w, reg_b):
    raise NotImplementedError("write your pallas kernel here")


# speedup vs baseline: 33.4674x; 33.4674x over previous
import functools
import math

import jax
import jax.numpy as jnp
from jax.experimental import pallas as pl

_SCALES = (128.0, 256.0, 512.0)
_RATIOS = (0.5, 1.0, 2.0)
_NMS_T = 0.7
_TOPK = 2000
_MIN_SIZE = 16.0
_LANES = 128
_BCLIP = math.log(1000.0 / 16)


def _gen_anchors(img_h, img_w, feat_h, feat_w):
    stride_h = img_h // feat_h
    stride_w = img_w // feat_w
    base = []
    for s in _SCALES:
        for r in _RATIOS:
            w = s * math.sqrt(1.0 / r)
            h = s * math.sqrt(r)
            base.append([-w / 2.0, -h / 2.0, w / 2.0, h / 2.0])
    base = jnp.asarray(base, dtype=jnp.float32)
    shifts_x = jnp.arange(feat_w, dtype=jnp.float32) * stride_w + stride_w // 2
    shifts_y = jnp.arange(feat_h, dtype=jnp.float32) * stride_h + stride_h // 2
    sy, sx = jnp.meshgrid(shifts_y, shifts_x, indexing='ij')
    sx = sx.reshape(-1)
    sy = sy.reshape(-1)
    shifts = jnp.stack([sx, sy, sx, sy], axis=1)
    return (shifts[:, None, :] + base[None, :, :]).reshape(-1, 4)


def _propose_nms_kernel(n, rows, img_h, img_w,
                        dx_ref, dy_ref, dw_ref, dh_ref,
                        ax1_ref, ay1_ref, ax2_ref, ay2_ref,
                        x1_out, y1_out, x2_out, y2_out, alive_out):
    aw = ax2_ref[...] - ax1_ref[...]
    ah = ay2_ref[...] - ay1_ref[...]
    acx = ax1_ref[...] + 0.5 * aw
    acy = ay1_ref[...] + 0.5 * ah
    dw = jnp.minimum(dw_ref[...], _BCLIP)
    dh = jnp.minimum(dh_ref[...], _BCLIP)
    px = dx_ref[...] * aw + acx
    py = dy_ref[...] * ah + acy
    pw = jnp.exp(dw) * aw
    ph = jnp.exp(dh) * ah
    x1 = jnp.clip(px - 0.5 * pw, 0.0, float(img_w))
    y1 = jnp.clip(py - 0.5 * ph, 0.0, float(img_h))
    x2 = jnp.clip(px + 0.5 * pw, 0.0, float(img_w))
    y2 = jnp.clip(py + 0.5 * ph, 0.0, float(img_h))
    x1_out[...] = x1
    y1_out[...] = y1
    x2_out[...] = x2
    y2_out[...] = y2

    ws = x2 - x1
    hs = y2 - y1
    areas = ws * hs
    valid = (ws >= _MIN_SIZE) & (hs >= _MIN_SIZE)
    alive_out[...] = valid.astype(jnp.float32)

    row_iota = jax.lax.broadcasted_iota(jnp.int32, (rows, _LANES), 0)
    lane_iota = jax.lax.broadcasted_iota(jnp.int32, (rows, _LANES), 1)
    flat_idx = row_iota * _LANES + lane_iota
    lane1 = jax.lax.broadcasted_iota(jnp.int32, (1, _LANES), 1)

    def body(i, _):
        r = i // _LANES
        c = i - r * _LANES
        onehot = (lane1 == c).astype(jnp.float32)
        bx1 = jnp.sum(x1_out[pl.ds(r, 1), :] * onehot)
        by1 = jnp.sum(y1_out[pl.ds(r, 1), :] * onehot)
        bx2 = jnp.sum(x2_out[pl.ds(r, 1), :] * onehot)
        by2 = jnp.sum(y2_out[pl.ds(r, 1), :] * onehot)
        alive_i = jnp.sum(alive_out[pl.ds(r, 1), :] * onehot)
        area_i = (bx2 - bx1) * (by2 - by1)
        xx1 = jnp.maximum(bx1, x1)
        yy1 = jnp.maximum(by1, y1)
        xx2 = jnp.minimum(bx2, x2)
        yy2 = jnp.minimum(by2, y2)
        inter = jnp.clip(xx2 - xx1, 0.0, None) * jnp.clip(yy2 - yy1, 0.0, None)
        iou = inter / (area_i + areas - inter)
        sup = (iou > _NMS_T) & (flat_idx > i) & (alive_i > 0.0)
        alive_out[...] = alive_out[...] * (1.0 - sup.astype(jnp.float32))
        return 0

    jax.lax.fori_loop(0, n, body, 0)


def kernel(image, feat, rpn_conv_w, rpn_conv_b, cls_w, cls_b, reg_w, reg_b):
    img_h, img_w = image.shape[-2], image.shape[-1]
    feat_h, feat_w = feat.shape[-2], feat.shape[-1]

    x = jax.lax.conv_general_dilated(
        feat, rpn_conv_w, window_strides=(1, 1), padding=[(1, 1), (1, 1)],
        dimension_numbers=('NCHW', 'OIHW', 'NCHW'))
    rpn_feat = jax.nn.relu(x + rpn_conv_b[None, :, None, None])
    cls = jax.lax.conv_general_dilated(
        rpn_feat, cls_w, window_strides=(1, 1), padding=[(0, 0), (0, 0)],
        dimension_numbers=('NCHW', 'OIHW', 'NCHW')) + cls_b[None, :, None, None]
    reg = jax.lax.conv_general_dilated(
        rpn_feat, reg_w, window_strides=(1, 1), padding=[(0, 0), (0, 0)],
        dimension_numbers=('NCHW', 'OIHW', 'NCHW')) + reg_b[None, :, None, None]

    b = cls.shape[0]
    a = cls.shape[1]
    anchors = _gen_anchors(img_h, img_w, feat_h, feat_w)
    cls_flat = jnp.transpose(cls, (0, 2, 3, 1)).reshape(-1)
    box_flat = jnp.transpose(
        reg.reshape(b, a, 4, feat_h, feat_w), (0, 3, 4, 1, 2)).reshape(-1, 4)
    scores = jax.nn.sigmoid(cls_flat)
    n = scores.shape[0]
    rows = n // _LANES
    top_scores, top_idx = jax.lax.top_k(scores, n)
    d = box_flat[top_idx]
    anc = anchors[top_idx]

    planes = [v.reshape(rows, _LANES) for v in
              (d[:, 0], d[:, 1], d[:, 2], d[:, 3],
               anc[:, 0], anc[:, 1], anc[:, 2], anc[:, 3])]
    plane = jax.ShapeDtypeStruct((rows, _LANES), jnp.float32)
    x1, y1, x2, y2, alive = pl.pallas_call(
        functools.partial(_propose_nms_kernel, n, rows, img_h, img_w),
        out_shape=[plane] * 5,
    )(*planes)

    props = jnp.stack(
        [x1.reshape(-1), y1.reshape(-1), x2.reshape(-1), y2.reshape(-1)], axis=1)
    kept = alive.reshape(-1) > 0.5
    rank = jnp.cumsum(kept.astype(jnp.int32)) - 1
    slot = jnp.where(kept & (rank < _TOPK), rank, _TOPK)
    out_props = jnp.zeros((_TOPK + 1, 4), jnp.float32).at[slot].set(props)[:_TOPK]
    out_scores = jnp.zeros((_TOPK + 1,), jnp.float32).at[slot].set(top_scores)[:_TOPK]
    return out_props, out_scores


# speedup vs baseline: 34.4581x; 1.0296x over previous
import functools
import math

import jax
import jax.numpy as jnp
from jax.experimental import pallas as pl

_SCALES = (128.0, 256.0, 512.0)
_RATIOS = (0.5, 1.0, 2.0)
_NMS_T = 0.7
_TOPK = 2000
_MIN_SIZE = 16.0
_LANES = 128
_BCLIP = math.log(1000.0 / 16)


def _gen_anchors(img_h, img_w, feat_h, feat_w):
    stride_h = img_h // feat_h
    stride_w = img_w // feat_w
    base = []
    for s in _SCALES:
        for r in _RATIOS:
            w = s * math.sqrt(1.0 / r)
            h = s * math.sqrt(r)
            base.append([-w / 2.0, -h / 2.0, w / 2.0, h / 2.0])
    base = jnp.asarray(base, dtype=jnp.float32)
    shifts_x = jnp.arange(feat_w, dtype=jnp.float32) * stride_w + stride_w // 2
    shifts_y = jnp.arange(feat_h, dtype=jnp.float32) * stride_h + stride_h // 2
    sy, sx = jnp.meshgrid(shifts_y, shifts_x, indexing='ij')
    sx = sx.reshape(-1)
    sy = sy.reshape(-1)
    shifts = jnp.stack([sx, sy, sx, sy], axis=1)
    return (shifts[:, None, :] + base[None, :, :]).reshape(-1, 4)


def _propose_nms_kernel(n, rows, img_h, img_w,
                        dx_ref, dy_ref, dw_ref, dh_ref,
                        ax1_ref, ay1_ref, ax2_ref, ay2_ref,
                        x1_out, y1_out, x2_out, y2_out, alive_out):
    aw = ax2_ref[...] - ax1_ref[...]
    ah = ay2_ref[...] - ay1_ref[...]
    acx = ax1_ref[...] + 0.5 * aw
    acy = ay1_ref[...] + 0.5 * ah
    dw = jnp.minimum(dw_ref[...], _BCLIP)
    dh = jnp.minimum(dh_ref[...], _BCLIP)
    px = dx_ref[...] * aw + acx
    py = dy_ref[...] * ah + acy
    pw = jnp.exp(dw) * aw
    ph = jnp.exp(dh) * ah
    x1 = jnp.clip(px - 0.5 * pw, 0.0, float(img_w))
    y1 = jnp.clip(py - 0.5 * ph, 0.0, float(img_h))
    x2 = jnp.clip(px + 0.5 * pw, 0.0, float(img_w))
    y2 = jnp.clip(py + 0.5 * ph, 0.0, float(img_h))
    x1_out[...] = x1
    y1_out[...] = y1
    x2_out[...] = x2
    y2_out[...] = y2

    ws = x2 - x1
    hs = y2 - y1
    areas = ws * hs
    valid = (ws >= _MIN_SIZE) & (hs >= _MIN_SIZE)
    alive_out[...] = valid.astype(jnp.float32)

    row_iota = jax.lax.broadcasted_iota(jnp.int32, (rows, _LANES), 0)
    lane_iota = jax.lax.broadcasted_iota(jnp.int32, (rows, _LANES), 1)
    flat_idx = row_iota * _LANES + lane_iota
    lane1 = jax.lax.broadcasted_iota(jnp.int32, (1, _LANES), 1)
    t_areas = _NMS_T * areas

    def body(i, _):
        r = i // _LANES
        c = i - r * _LANES
        onehot = (lane1 == c).astype(jnp.float32)
        alive_i = jnp.sum(alive_out[pl.ds(r, 1), :] * onehot)

        @pl.when(alive_i > 0.0)
        def _():
            bx1 = jnp.sum(x1_out[pl.ds(r, 1), :] * onehot)
            by1 = jnp.sum(y1_out[pl.ds(r, 1), :] * onehot)
            bx2 = jnp.sum(x2_out[pl.ds(r, 1), :] * onehot)
            by2 = jnp.sum(y2_out[pl.ds(r, 1), :] * onehot)
            area_i = (bx2 - bx1) * (by2 - by1)
            xx1 = jnp.maximum(bx1, x1)
            yy1 = jnp.maximum(by1, y1)
            xx2 = jnp.minimum(bx2, x2)
            yy2 = jnp.minimum(by2, y2)
            inter = jnp.clip(xx2 - xx1, 0.0, None) * jnp.clip(yy2 - yy1, 0.0, None)
            sup = (inter * (1.0 + _NMS_T) > _NMS_T * area_i + t_areas) \
                & (flat_idx > i)
            alive_out[...] = alive_out[...] * (1.0 - sup.astype(jnp.float32))

        return 0

    jax.lax.fori_loop(0, n, body, 0)


def kernel(image, feat, rpn_conv_w, rpn_conv_b, cls_w, cls_b, reg_w, reg_b):
    img_h, img_w = image.shape[-2], image.shape[-1]
    feat_h, feat_w = feat.shape[-2], feat.shape[-1]

    x = jax.lax.conv_general_dilated(
        feat, rpn_conv_w, window_strides=(1, 1), padding=[(1, 1), (1, 1)],
        dimension_numbers=('NCHW', 'OIHW', 'NCHW'))
    rpn_feat = jax.nn.relu(x + rpn_conv_b[None, :, None, None])
    cls = jax.lax.conv_general_dilated(
        rpn_feat, cls_w, window_strides=(1, 1), padding=[(0, 0), (0, 0)],
        dimension_numbers=('NCHW', 'OIHW', 'NCHW')) + cls_b[None, :, None, None]
    reg = jax.lax.conv_general_dilated(
        rpn_feat, reg_w, window_strides=(1, 1), padding=[(0, 0), (0, 0)],
        dimension_numbers=('NCHW', 'OIHW', 'NCHW')) + reg_b[None, :, None, None]

    b = cls.shape[0]
    a = cls.shape[1]
    anchors = _gen_anchors(img_h, img_w, feat_h, feat_w)
    cls_flat = jnp.transpose(cls, (0, 2, 3, 1)).reshape(-1)
    box_flat = jnp.transpose(
        reg.reshape(b, a, 4, feat_h, feat_w), (0, 3, 4, 1, 2)).reshape(-1, 4)
    scores = jax.nn.sigmoid(cls_flat)
    n = scores.shape[0]
    rows = n // _LANES
    top_scores, top_idx = jax.lax.top_k(scores, n)
    d = box_flat[top_idx]
    anc = anchors[top_idx]

    planes = [v.reshape(rows, _LANES) for v in
              (d[:, 0], d[:, 1], d[:, 2], d[:, 3],
               anc[:, 0], anc[:, 1], anc[:, 2], anc[:, 3])]
    plane = jax.ShapeDtypeStruct((rows, _LANES), jnp.float32)
    x1, y1, x2, y2, alive = pl.pallas_call(
        functools.partial(_propose_nms_kernel, n, rows, img_h, img_w),
        out_shape=[plane] * 5,
    )(*planes)

    props = jnp.stack(
        [x1.reshape(-1), y1.reshape(-1), x2.reshape(-1), y2.reshape(-1)], axis=1)
    kept = alive.reshape(-1) > 0.5
    rank = jnp.cumsum(kept.astype(jnp.int32)) - 1
    slot = jnp.where(kept & (rank < _TOPK), rank, _TOPK)
    out_props = jnp.zeros((_TOPK + 1, 4), jnp.float32).at[slot].set(props)[:_TOPK]
    out_scores = jnp.zeros((_TOPK + 1,), jnp.float32).at[slot].set(top_scores)[:_TOPK]
    return out_props, out_scores


# speedup vs baseline: 34.6243x; 1.0048x over previous
import functools
import math

import jax
import jax.numpy as jnp
from jax.experimental import pallas as pl

_SCALES = (128.0, 256.0, 512.0)
_RATIOS = (0.5, 1.0, 2.0)
_NMS_T = 0.7
_TOPK = 2000
_MIN_SIZE = 16.0
_LANES = 128
_BCLIP = math.log(1000.0 / 16)


def _gen_anchors(img_h, img_w, feat_h, feat_w):
    stride_h = img_h // feat_h
    stride_w = img_w // feat_w
    base = []
    for s in _SCALES:
        for r in _RATIOS:
            w = s * math.sqrt(1.0 / r)
            h = s * math.sqrt(r)
            base.append([-w / 2.0, -h / 2.0, w / 2.0, h / 2.0])
    base = jnp.asarray(base, dtype=jnp.float32)
    shifts_x = jnp.arange(feat_w, dtype=jnp.float32) * stride_w + stride_w // 2
    shifts_y = jnp.arange(feat_h, dtype=jnp.float32) * stride_h + stride_h // 2
    sy, sx = jnp.meshgrid(shifts_y, shifts_x, indexing='ij')
    sx = sx.reshape(-1)
    sy = sy.reshape(-1)
    shifts = jnp.stack([sx, sy, sx, sy], axis=1)
    return (shifts[:, None, :] + base[None, :, :]).reshape(-1, 4)


def _propose_nms_kernel(n, rows, img_h, img_w,
                        dx_ref, dy_ref, dw_ref, dh_ref,
                        ax1_ref, ay1_ref, ax2_ref, ay2_ref,
                        x1_out, y1_out, x2_out, y2_out, alive_out):
    aw = ax2_ref[...] - ax1_ref[...]
    ah = ay2_ref[...] - ay1_ref[...]
    acx = ax1_ref[...] + 0.5 * aw
    acy = ay1_ref[...] + 0.5 * ah
    dw = jnp.minimum(dw_ref[...], _BCLIP)
    dh = jnp.minimum(dh_ref[...], _BCLIP)
    px = dx_ref[...] * aw + acx
    py = dy_ref[...] * ah + acy
    pw = jnp.exp(dw) * aw
    ph = jnp.exp(dh) * ah
    x1 = jnp.clip(px - 0.5 * pw, 0.0, float(img_w))
    y1 = jnp.clip(py - 0.5 * ph, 0.0, float(img_h))
    x2 = jnp.clip(px + 0.5 * pw, 0.0, float(img_w))
    y2 = jnp.clip(py + 0.5 * ph, 0.0, float(img_h))
    x1_out[...] = x1
    y1_out[...] = y1
    x2_out[...] = x2
    y2_out[...] = y2

    ws = x2 - x1
    hs = y2 - y1
    areas = ws * hs
    valid = (ws >= _MIN_SIZE) & (hs >= _MIN_SIZE)
    alive_out[...] = valid.astype(jnp.float32)

    row_iota = jax.lax.broadcasted_iota(jnp.int32, (rows, _LANES), 0)
    lane_iota = jax.lax.broadcasted_iota(jnp.int32, (rows, _LANES), 1)
    flat_idx = row_iota * _LANES + lane_iota
    lane1 = jax.lax.broadcasted_iota(jnp.int32, (1, _LANES), 1)
    t_areas = _NMS_T * areas

    band = 8
    for q in range(rows // band):
        s = q * band
        x1s, y1s, x2s, y2s = x1[s:], y1[s:], x2[s:], y2[s:]
        t_areas_s = t_areas[s:]
        flat_idx_s = flat_idx[s:]

        def body(i, _, s=s, x1s=x1s, y1s=y1s, x2s=x2s, y2s=y2s,
                 t_areas_s=t_areas_s, flat_idx_s=flat_idx_s):
            r = i // _LANES
            c = i - r * _LANES
            onehot = (lane1 == c).astype(jnp.float32)
            alive_i = jnp.sum(alive_out[pl.ds(r, 1), :] * onehot)

            @pl.when(alive_i > 0.0)
            def _():
                bx1 = jnp.sum(x1_out[pl.ds(r, 1), :] * onehot)
                by1 = jnp.sum(y1_out[pl.ds(r, 1), :] * onehot)
                bx2 = jnp.sum(x2_out[pl.ds(r, 1), :] * onehot)
                by2 = jnp.sum(y2_out[pl.ds(r, 1), :] * onehot)
                area_i = (bx2 - bx1) * (by2 - by1)
                xx1 = jnp.maximum(bx1, x1s)
                yy1 = jnp.maximum(by1, y1s)
                xx2 = jnp.minimum(bx2, x2s)
                yy2 = jnp.minimum(by2, y2s)
                inter = (jnp.clip(xx2 - xx1, 0.0, None)
                         * jnp.clip(yy2 - yy1, 0.0, None))
                sup = (inter * (1.0 + _NMS_T) > _NMS_T * area_i + t_areas_s) \
                    & (flat_idx_s > i)
                alive_out[s:, :] = (alive_out[s:, :]
                                    * (1.0 - sup.astype(jnp.float32)))

            return 0

        jax.lax.fori_loop(s * _LANES, (s + band) * _LANES, body, 0)


def kernel(image, feat, rpn_conv_w, rpn_conv_b, cls_w, cls_b, reg_w, reg_b):
    img_h, img_w = image.shape[-2], image.shape[-1]
    feat_h, feat_w = feat.shape[-2], feat.shape[-1]

    x = jax.lax.conv_general_dilated(
        feat, rpn_conv_w, window_strides=(1, 1), padding=[(1, 1), (1, 1)],
        dimension_numbers=('NCHW', 'OIHW', 'NCHW'))
    rpn_feat = jax.nn.relu(x + rpn_conv_b[None, :, None, None])
    cls = jax.lax.conv_general_dilated(
        rpn_feat, cls_w, window_strides=(1, 1), padding=[(0, 0), (0, 0)],
        dimension_numbers=('NCHW', 'OIHW', 'NCHW')) + cls_b[None, :, None, None]
    reg = jax.lax.conv_general_dilated(
        rpn_feat, reg_w, window_strides=(1, 1), padding=[(0, 0), (0, 0)],
        dimension_numbers=('NCHW', 'OIHW', 'NCHW')) + reg_b[None, :, None, None]

    b = cls.shape[0]
    a = cls.shape[1]
    anchors = _gen_anchors(img_h, img_w, feat_h, feat_w)
    cls_flat = jnp.transpose(cls, (0, 2, 3, 1)).reshape(-1)
    box_flat = jnp.transpose(
        reg.reshape(b, a, 4, feat_h, feat_w), (0, 3, 4, 1, 2)).reshape(-1, 4)
    scores = jax.nn.sigmoid(cls_flat)
    n = scores.shape[0]
    rows = n // _LANES
    top_scores, top_idx = jax.lax.top_k(scores, n)
    d = box_flat[top_idx]
    anc = anchors[top_idx]

    planes = [v.reshape(rows, _LANES) for v in
              (d[:, 0], d[:, 1], d[:, 2], d[:, 3],
               anc[:, 0], anc[:, 1], anc[:, 2], anc[:, 3])]
    plane = jax.ShapeDtypeStruct((rows, _LANES), jnp.float32)
    x1, y1, x2, y2, alive = pl.pallas_call(
        functools.partial(_propose_nms_kernel, n, rows, img_h, img_w),
        out_shape=[plane] * 5,
    )(*planes)

    props = jnp.stack(
        [x1.reshape(-1), y1.reshape(-1), x2.reshape(-1), y2.reshape(-1)], axis=1)
    kept = alive.reshape(-1) > 0.5
    rank = jnp.cumsum(kept.astype(jnp.int32)) - 1
    slot = jnp.where(kept & (rank < _TOPK), rank, _TOPK)
    out_props = jnp.zeros((_TOPK + 1, 4), jnp.float32).at[slot].set(props)[:_TOPK]
    out_scores = jnp.zeros((_TOPK + 1,), jnp.float32).at[slot].set(top_scores)[:_TOPK]
    return out_props, out_scores
